# baseline (speedup 1.0000x reference)
import jax
import jax.numpy as jnp
from jax import lax
from jax.experimental import pallas as pl
from jax.experimental.pallas import tpu as pltpu
from jax.experimental.pallas import tpu_sc as plsc

DIM = 1024
N_MARKERS = 3
NW = 32
CH = 16
NCH = 16
NB = 3
TPW = CH * NCH
NG = TPW // 16


def _body(tok_hbm, table_hbm, meta_hbm, out_hbm,
          tok_v, meta_v, buf0, buf1, buf2, sp,
          gsem0, gsem1, gsem2, csem0, csem1, csem2,
          osem0, osem1, osem2, msem):
    c = lax.axis_index("c")
    s = lax.axis_index("s")
    wid = s * 2 + c
    base = wid * TPW
    ncols = tok_hbm.shape[1]
    row = (wid * TPW) // ncols
    col = pl.multiple_of((wid * TPW) % ncols, TPW)

    pltpu.sync_copy(tok_hbm.at[row, pl.ds(col, TPW)], tok_v)
    meta_cp = pltpu.async_copy(meta_hbm, meta_v, msem)

    lanes = lax.iota(jnp.int32, 16)

    def lane_min(v):
        for sh in (1, 2, 4, 8):
            v = jnp.minimum(v, v.at[lanes ^ sh].get(mode="promise_in_bounds"))
        return v

    def min_body(g, m):
        off = pl.multiple_of(g * 16, 16)
        return jnp.minimum(m, tok_v[pl.ds(off, 16)])

    gm = lane_min(lax.fori_loop(1, NG, min_body, tok_v[pl.ds(0, 16)]))

    bufs = (buf0, buf1, buf2)
    gsems = (gsem0, gsem1, gsem2)
    csems = (csem0, csem1, csem2)
    osems = (osem0, osem1, osem2)
    slab = s * (NB * CH)

    def gather(j):
        return pltpu.async_copy(
            table_hbm.at[tok_v.at[pl.ds(j * CH, CH)]], bufs[j % NB],
            gsems[j % NB])

    def to_spmem(j, b):
        return pltpu.async_copy(
            bufs[b], sp.at[pl.ds(slab + b * CH, CH)], csems[b])

    def to_hbm(j, b):
        return pltpu.async_copy(
            sp.at[pl.ds(slab + b * CH, CH)],
            out_hbm.at[pl.ds(base + j * CH, CH)], osems[b])

    gh = [None] * NCH
    ch = [None] * NB
    oh = [None] * NB
    for j in range(NB - 1):
        gh[j] = gather(j)
    for j in range(NCH):
        b = j % NB
        jn = j + NB - 1
        if jn < NCH:
            nb = jn % NB
            if ch[nb] is not None:
                ch[nb].wait()
                oh[nb] = to_hbm(j - 1, nb)
            gh[jn] = gather(jn)
        gh[j].wait()
        if oh[b] is not None:
            oh[b].wait()
        ch[b] = to_spmem(j, b)
    for t in range(NCH - NB, NCH):
        tb = t % NB
        ch[tb].wait()
        oh[tb] = to_hbm(t, tb)
    for h in oh:
        h.wait()

    meta_cp.wait()

    @pl.when(gm[0] < N_MARKERS)
    def _():
        def group_body(g, carry):
            off = pl.multiple_of(g * 16, 16)
            t16 = tok_v[pl.ds(off, 16)]

            @pl.when(lane_min(t16)[0] < N_MARKERS)
            def _():
                def fix(l, c2):
                    sel = (lanes + l) & 15
                    t = t16.at[sel].get(mode="promise_in_bounds")[0]

                    @pl.when(t < N_MARKERS)
                    def __():
                        pltpu.sync_copy(
                            meta_v.at[t],
                            out_hbm.at[base + g * 16 + l])
                    return c2

                lax.fori_loop(0, 16, fix, 0)
            return carry

        lax.fori_loop(0, NG, group_body, 0)


@jax.jit
def _run(tokens, table, meta_reps):
    return pl.kernel(
        _body,
        out_type=jax.ShapeDtypeStruct((NW * TPW, DIM), jnp.float32),
        mesh=plsc.VectorSubcoreMesh(core_axis_name="c", subcore_axis_name="s"),
        scratch_types=[
            pltpu.VMEM((TPW,), jnp.int32),
            pltpu.VMEM((N_MARKERS, DIM), jnp.float32),
            pltpu.VMEM((CH, DIM), jnp.float32),
            pltpu.VMEM((CH, DIM), jnp.float32),
            pltpu.VMEM((CH, DIM), jnp.float32),
            pltpu.VMEM_SHARED((16 * NB * CH, DIM), jnp.float32),
            pltpu.SemaphoreType.DMA,
            pltpu.SemaphoreType.DMA,
            pltpu.SemaphoreType.DMA,
            pltpu.SemaphoreType.DMA,
            pltpu.SemaphoreType.DMA,
            pltpu.SemaphoreType.DMA,
            pltpu.SemaphoreType.DMA,
            pltpu.SemaphoreType.DMA,
            pltpu.SemaphoreType.DMA,
            pltpu.SemaphoreType.DMA,
        ],
    )(tokens, table, meta_reps)


def kernel(tokens, table, meta_reps):
    b, s = tokens.shape
    out = _run(tokens, table, meta_reps)
    return out.reshape(b, s, DIM)


# speedup vs baseline: 1.0401x; 1.0401x over previous
import jax
import jax.numpy as jnp
from jax import lax
from jax.experimental import pallas as pl
from jax.experimental.pallas import tpu as pltpu
from jax.experimental.pallas import tpu_sc as plsc

DIM = 1024
N_MARKERS = 3
NW = 32
CH = 32
NCH = 8
NB = 3
TPW = CH * NCH
NG = TPW // 16


def _body(tok_hbm, table_hbm, meta_hbm, out_hbm,
          tok_v, meta_v, buf0, buf1, buf2,
          gsem0, gsem1, gsem2, osem0, osem1, osem2, msem):
    c = lax.axis_index("c")
    s = lax.axis_index("s")
    wid = s * 2 + c
    base = wid * TPW
    ncols = tok_hbm.shape[1]
    row = (wid * TPW) // ncols
    col = pl.multiple_of((wid * TPW) % ncols, TPW)

    pltpu.sync_copy(tok_hbm.at[row, pl.ds(col, TPW)], tok_v)
    meta_cp = pltpu.async_copy(meta_hbm, meta_v, msem)

    lanes = lax.iota(jnp.int32, 16)

    def lane_min(v):
        for sh in (1, 2, 4, 8):
            v = jnp.minimum(v, v.at[lanes ^ sh].get(mode="promise_in_bounds"))
        return v

    def min_body(g, m):
        off = pl.multiple_of(g * 16, 16)
        return jnp.minimum(m, tok_v[pl.ds(off, 16)])

    gm = lane_min(lax.fori_loop(1, NG, min_body, tok_v[pl.ds(0, 16)]))

    bufs = (buf0, buf1, buf2)
    gsems = (gsem0, gsem1, gsem2)
    osems = (osem0, osem1, osem2)

    def gather(j):
        return pltpu.async_copy(
            table_hbm.at[tok_v.at[pl.ds(j * CH, CH)]], bufs[j % NB],
            gsems[j % NB])

    gh = [None] * NCH
    oh = [None] * NB
    for j in range(NB - 1):
        gh[j] = gather(j)
    for j in range(NCH):
        b = j % NB
        jn = j + NB - 1
        if jn < NCH:
            nb = jn % NB
            if oh[nb] is not None:
                oh[nb].wait()
            gh[jn] = gather(jn)
        gh[j].wait()
        oh[b] = pltpu.async_copy(
            bufs[b], out_hbm.at[pl.ds(base + j * CH, CH)], osems[b])
    for h in oh:
        h.wait()

    meta_cp.wait()

    @pl.when(gm[0] < N_MARKERS)
    def _():
        def group_body(g, carry):
            off = pl.multiple_of(g * 16, 16)
            t16 = tok_v[pl.ds(off, 16)]

            @pl.when(lane_min(t16)[0] < N_MARKERS)
            def _():
                def fix(l, c2):
                    sel = (lanes + l) & 15
                    t = t16.at[sel].get(mode="promise_in_bounds")[0]

                    @pl.when(t < N_MARKERS)
                    def __():
                        pltpu.sync_copy(
                            meta_v.at[t],
                            out_hbm.at[base + g * 16 + l])
                    return c2

                lax.fori_loop(0, 16, fix, 0)
            return carry

        lax.fori_loop(0, NG, group_body, 0)


@jax.jit
def _run(tokens, table, meta_reps):
    return pl.kernel(
        _body,
        out_type=jax.ShapeDtypeStruct((NW * TPW, DIM), jnp.float32),
        mesh=plsc.VectorSubcoreMesh(core_axis_name="c", subcore_axis_name="s"),
        scratch_types=[
            pltpu.VMEM((TPW,), jnp.int32),
            pltpu.VMEM((N_MARKERS, DIM), jnp.float32),
            pltpu.VMEM((CH, DIM), jnp.float32),
            pltpu.VMEM((CH, DIM), jnp.float32),
            pltpu.VMEM((CH, DIM), jnp.float32),
            pltpu.SemaphoreType.DMA,
            pltpu.SemaphoreType.DMA,
            pltpu.SemaphoreType.DMA,
            pltpu.SemaphoreType.DMA,
            pltpu.SemaphoreType.DMA,
            pltpu.SemaphoreType.DMA,
            pltpu.SemaphoreType.DMA,
        ],
    )(tokens, table, meta_reps)


def kernel(tokens, table, meta_reps):
    b, s = tokens.shape
    out = _run(tokens, table, meta_reps)
    return out.reshape(b, s, DIM)
